# baseline (speedup 1.0000x reference)
import jax
import jax.numpy as jnp
from jax.experimental import pallas as pl
from jax.experimental.pallas import tpu as pltpu

_NF = 128
_HI = jax.lax.Precision.HIGHEST


def _sel128(vec_ref, idx):
    v = vec_ref[...].reshape(1, _NF)
    lanes = jax.lax.broadcasted_iota(jnp.int32, (1, _NF), 1)
    return jnp.sum(jnp.where(lanes == idx % _NF, v, 0.0))


def _tc_body(user_p, att_p, item_p, uf_ref, tf_ref, itf_ref,
             b_ref0, b_ref1, b_ref2, out_ref):
    u0 = user_p[0]
    a0 = att_p[0]
    i0 = item_p[0]
    ur = u0 % 8
    ar = a0 % 8
    ir = i0 % 8

    rows = jax.lax.broadcasted_iota(jnp.int32, (8, _NF), 0)
    u_sel = jnp.sum(jnp.where(rows == ur, uf_ref[...], 0.0), axis=0,
                    keepdims=True)
    i_sel = jnp.sum(jnp.where(rows == ir, itf_ref[...], 0.0), axis=0,
                    keepdims=True)

    ident = (jax.lax.broadcasted_iota(jnp.int32, (_NF, _NF), 0)
             == jax.lax.broadcasted_iota(jnp.int32, (_NF, _NF), 1)
             ).astype(jnp.float32)
    u_col = jax.lax.dot_general(ident, u_sel, (((1,), (1,)), ((), ())),
                                precision=_HI,
                                preferred_element_type=jnp.float32)
    outer = jax.lax.dot_general(u_col, i_sel, (((1,), (0,)), ((), ())),
                                precision=_HI,
                                preferred_element_type=jnp.float32)

    s8 = jnp.zeros((8, _NF), jnp.float32)
    for a in range(_NF):
        s8 = s8 + tf_ref[:, pl.ds(a * _NF, _NF)] * outer[a:a + 1, :]
    pred = jnp.sum(jnp.where(rows == ar, s8, 0.0))

    pred = (pred + _sel128(b_ref0, a0) + _sel128(b_ref1, 200 + u0)
            + _sel128(b_ref2, 100200 + i0))
    out_ref[0, 0] = 1.0 / (1.0 + jnp.exp(-pred))


def _tc_call(u32, a32, i32, uf, tf, itf, biases):
    grid_spec = pltpu.PrefetchScalarGridSpec(
        num_scalar_prefetch=3,
        grid=(1,),
        in_specs=[
            pl.BlockSpec((8, _NF), lambda g, u, a, i: (u[0] // 8, 0)),
            pl.BlockSpec((8, 16384), lambda g, u, a, i: (a[0] // 8, 0)),
            pl.BlockSpec((8, _NF), lambda g, u, a, i: (i[0] // 8, 0)),
            pl.BlockSpec((_NF,), lambda g, u, a, i: (a[0] // _NF,)),
            pl.BlockSpec((_NF,), lambda g, u, a, i: ((200 + u[0]) // _NF,)),
            pl.BlockSpec((_NF,), lambda g, u, a, i: ((100200 + i[0]) // _NF,)),
        ],
        out_specs=pl.BlockSpec((1, 1), lambda g, u, a, i: (0, 0),
                               memory_space=pltpu.SMEM),
    )
    out = pl.pallas_call(
        _tc_body, grid_spec=grid_spec,
        out_shape=jax.ShapeDtypeStruct((1, 1), jnp.float32),
    )(u32, a32, i32, uf, tf, itf, biases, biases, biases)
    return out.reshape(1)


def kernel(user, attempt, item, view, user_factors, time_factors, item_factors,
           stress_item_factor, time_biases, stress_user_biases,
           stress_item_biases, rate_user_biases, rate_item_biases,
           done_user_biases, done_item_biases):
    del view, stress_item_factor, stress_user_biases, stress_item_biases
    del rate_user_biases, rate_item_biases
    biases = jnp.concatenate([
        time_biases.reshape(-1), done_user_biases.reshape(-1),
        done_item_biases.reshape(-1)])
    return _tc_call(user.astype(jnp.int32), attempt.astype(jnp.int32),
                    item.astype(jnp.int32), user_factors, time_factors,
                    item_factors, biases)


# speedup vs baseline: 2.5073x; 2.5073x over previous
import jax
import jax.numpy as jnp
from jax.experimental import pallas as pl
from jax.experimental.pallas import tpu as pltpu

_NF = 128
_HI = jax.lax.Precision.HIGHEST


def _sel128(vec_ref, idx, n):
    start = jnp.minimum(idx // _NF * _NF, n - _NF)
    lanes = jax.lax.broadcasted_iota(jnp.int32, (1, _NF), 1)
    return jnp.sum(jnp.where(lanes == idx - start, vec_ref[...], 0.0))


def _tc_body(user_p, att_p, item_p, uf_ref, tf_ref, itf_ref,
             tb_ref, dub_ref, dib_ref, out_ref):
    u0 = user_p[0]
    a0 = att_p[0]
    i0 = item_p[0]
    ur = u0 % 8
    ar = a0 % 8
    ir = i0 % 8

    rows = jax.lax.broadcasted_iota(jnp.int32, (8, _NF), 0)
    u_sel = jnp.sum(jnp.where(rows == ur, uf_ref[...], 0.0), axis=0,
                    keepdims=True)
    i_sel = jnp.sum(jnp.where(rows == ir, itf_ref[...], 0.0), axis=0,
                    keepdims=True)

    ident = (jax.lax.broadcasted_iota(jnp.int32, (_NF, _NF), 0)
             == jax.lax.broadcasted_iota(jnp.int32, (_NF, _NF), 1)
             ).astype(jnp.float32)
    u_col = jax.lax.dot_general(ident, u_sel, (((1,), (1,)), ((), ())),
                                precision=_HI,
                                preferred_element_type=jnp.float32)
    outer = jax.lax.dot_general(u_col, i_sel, (((1,), (0,)), ((), ())),
                                precision=_HI,
                                preferred_element_type=jnp.float32)

    s8 = jnp.zeros((8, _NF), jnp.float32)
    for a in range(_NF):
        s8 = s8 + tf_ref[:, pl.ds(a * _NF, _NF)] * outer[a:a + 1, :]
    pred = jnp.sum(jnp.where(rows == ar, s8, 0.0))

    pred = (pred + _sel128(tb_ref, a0, 200) + _sel128(dub_ref, u0, 100000)
            + _sel128(dib_ref, i0, 100000))
    out_ref[0, 0] = 1.0 / (1.0 + jnp.exp(-pred))


def _tc_call(u32, a32, i32, uf, tf, itf, tb, dub, dib):
    grid_spec = pltpu.PrefetchScalarGridSpec(
        num_scalar_prefetch=3,
        grid=(1,),
        in_specs=[
            pl.BlockSpec((8, _NF), lambda g, u, a, i: (u[0] // 8, 0)),
            pl.BlockSpec((8, 16384), lambda g, u, a, i: (a[0] // 8, 0)),
            pl.BlockSpec((8, _NF), lambda g, u, a, i: (i[0] // 8, 0)),
            pl.BlockSpec((1, _NF), lambda g, u, a, i: (0, 0)),
            pl.BlockSpec((1, _NF), lambda g, u, a, i: (0, 0)),
            pl.BlockSpec((1, _NF), lambda g, u, a, i: (0, 0)),
        ],
        out_specs=pl.BlockSpec((1, 1), lambda g, u, a, i: (0, 0),
                               memory_space=pltpu.SMEM),
    )
    out = pl.pallas_call(
        _tc_body, grid_spec=grid_spec,
        out_shape=jax.ShapeDtypeStruct((1, 1), jnp.float32),
    )(u32, a32, i32, uf, tf, itf, tb, dub, dib)
    return out.reshape(1)


def kernel(user, attempt, item, view, user_factors, time_factors, item_factors,
           stress_item_factor, time_biases, stress_user_biases,
           stress_item_biases, rate_user_biases, rate_item_biases,
           done_user_biases, done_item_biases):
    del view, stress_item_factor, stress_user_biases, stress_item_biases
    del rate_user_biases, rate_item_biases
    a32 = attempt.astype(jnp.int32)
    u32 = user.astype(jnp.int32)
    i32 = item.astype(jnp.int32)
    tbw = jax.lax.dynamic_slice(time_biases, (a32[0] // _NF * _NF, 0),
                                (_NF, 1)).reshape(1, _NF)
    dubw = jax.lax.dynamic_slice(done_user_biases, (u32[0] // _NF * _NF, 0),
                                 (_NF, 1)).reshape(1, _NF)
    dibw = jax.lax.dynamic_slice(done_item_biases, (i32[0] // _NF * _NF, 0),
                                 (_NF, 1)).reshape(1, _NF)
    return _tc_call(u32, a32, i32, user_factors, time_factors,
                    item_factors, tbw, dubw, dibw)


# speedup vs baseline: 2.7448x; 1.0947x over previous
import jax
import jax.numpy as jnp
from jax.experimental import pallas as pl
from jax.experimental.pallas import tpu as pltpu

_NF = 128
_HI = jax.lax.Precision.HIGHEST


def _sel128(vec_ref, idx, n):
    start = jnp.minimum(idx // _NF * _NF, n - _NF)
    lanes = jax.lax.broadcasted_iota(jnp.int32, (1, _NF), 1)
    return jnp.sum(jnp.where(lanes == idx - start, vec_ref[...], 0.0))


def _tc_body(user_p, att_p, item_p, uf_ref, tf_ref, itf_ref,
             bw_ref, out_ref):
    u0 = user_p[0]
    a0 = att_p[0]
    i0 = item_p[0]
    ur = u0 % 8
    ar = a0 % 8
    ir = i0 % 8

    rows = jax.lax.broadcasted_iota(jnp.int32, (8, _NF), 0)
    u_sel = jnp.sum(jnp.where(rows == ur, uf_ref[...], 0.0), axis=0,
                    keepdims=True)
    i_sel = jnp.sum(jnp.where(rows == ir, itf_ref[...], 0.0), axis=0,
                    keepdims=True)

    ident = (jax.lax.broadcasted_iota(jnp.int32, (_NF, _NF), 0)
             == jax.lax.broadcasted_iota(jnp.int32, (_NF, _NF), 1)
             ).astype(jnp.float32)
    u_col = jax.lax.dot_general(ident, u_sel, (((1,), (1,)), ((), ())),
                                precision=_HI,
                                preferred_element_type=jnp.float32)
    outer = jax.lax.dot_general(u_col, i_sel, (((1,), (0,)), ((), ())),
                                precision=_HI,
                                preferred_element_type=jnp.float32)

    s8 = jnp.zeros((8, _NF), jnp.float32)
    for a in range(_NF):
        s8 = s8 + tf_ref[:, pl.ds(a * _NF, _NF)] * outer[a:a + 1, :]
    pred = jnp.sum(jnp.where(rows == ar, s8, 0.0))

    pred = (pred + _sel128(bw_ref.at[0:1, :], a0, 200)
            + _sel128(bw_ref.at[1:2, :], u0, 100000)
            + _sel128(bw_ref.at[2:3, :], i0, 100000))
    out_ref[0, 0] = 1.0 / (1.0 + jnp.exp(-pred))


def _tc_call(u32, a32, i32, uf, tf, itf, bw):
    grid_spec = pltpu.PrefetchScalarGridSpec(
        num_scalar_prefetch=3,
        grid=(1,),
        in_specs=[
            pl.BlockSpec((8, _NF), lambda g, u, a, i: (u[0] // 8, 0)),
            pl.BlockSpec((8, 16384), lambda g, u, a, i: (a[0] // 8, 0)),
            pl.BlockSpec((8, _NF), lambda g, u, a, i: (i[0] // 8, 0)),
            pl.BlockSpec((3, _NF), lambda g, u, a, i: (0, 0)),
        ],
        out_specs=pl.BlockSpec((1, 1), lambda g, u, a, i: (0, 0),
                               memory_space=pltpu.SMEM),
    )
    out = pl.pallas_call(
        _tc_body, grid_spec=grid_spec,
        out_shape=jax.ShapeDtypeStruct((1, 1), jnp.float32),
    )(u32, a32, i32, uf, tf, itf, bw)
    return out.reshape(1)


def kernel(user, attempt, item, view, user_factors, time_factors, item_factors,
           stress_item_factor, time_biases, stress_user_biases,
           stress_item_biases, rate_user_biases, rate_item_biases,
           done_user_biases, done_item_biases):
    del view, stress_item_factor, stress_user_biases, stress_item_biases
    del rate_user_biases, rate_item_biases
    a32 = attempt.astype(jnp.int32)
    u32 = user.astype(jnp.int32)
    i32 = item.astype(jnp.int32)
    tbw = jax.lax.dynamic_slice(time_biases, (a32[0] // _NF * _NF, 0),
                                (_NF, 1)).reshape(1, _NF)
    dubw = jax.lax.dynamic_slice(done_user_biases, (u32[0] // _NF * _NF, 0),
                                 (_NF, 1)).reshape(1, _NF)
    dibw = jax.lax.dynamic_slice(done_item_biases, (i32[0] // _NF * _NF, 0),
                                 (_NF, 1)).reshape(1, _NF)
    bw = jnp.concatenate([tbw, dubw, dibw], axis=0)
    return _tc_call(u32, a32, i32, user_factors, time_factors,
                    item_factors, bw)
